# Initial kernel scaffold; baseline (speedup 1.0000x reference)
import jax
import jax.numpy as jnp
from jax.experimental import pallas as pl


def kernel(tokens, table):
    raise NotImplementedError("write your pallas kernel here")


# speedup vs baseline: 1.3988x; 1.3988x over previous
import functools
import math

import jax
import jax.numpy as jnp
from jax import lax
from jax.experimental import pallas as pl
from jax.experimental.pallas import tpu as pltpu
from jax.experimental.pallas import tpu_sc as plsc

EMBED = 32
SCALE = math.sqrt(float(EMBED))
LANES = 16


def _emb_call(B, NC, NS):
    NW = NC * NS
    b_per_w = B // NW
    CH = 1024
    n_ch = b_per_w // CH
    assert n_ch * CH == b_per_w

    mesh = plsc.VectorSubcoreMesh(core_axis_name="c", subcore_axis_name="s")

    @functools.partial(
        pl.kernel,
        mesh=mesh,
        out_type=jax.ShapeDtypeStruct((B, EMBED), jnp.float32),
        compiler_params=pltpu.CompilerParams(use_tc_tiling_on_sc=False),
        scratch_types=[
            pltpu.VMEM((CH,), jnp.int32),
            pltpu.VMEM((CH, EMBED), jnp.float32),
            pltpu.SemaphoreType.DMA,
        ],
    )
    def emb_kernel(idx_hbm, table_hbm, out_hbm, idx_v, rows_v, sem):
        wid = lax.axis_index("s") * NC + lax.axis_index("c")
        base = wid * b_per_w

        def chunk_body(c, carry):
            off = base + c * CH
            pltpu.sync_copy(idx_hbm.at[pl.ds(off, CH)], idx_v)
            pltpu.async_copy(table_hbm.at[idx_v], rows_v, sem).wait()

            def scale_body(i, carry2):
                for j in range(EMBED // LANES):
                    sl = pl.ds(j * LANES, LANES)
                    rows_v[i, sl] = rows_v[i, sl] * SCALE
                return carry2

            lax.fori_loop(0, CH, scale_body, 0, unroll=4)
            pltpu.sync_copy(rows_v, out_hbm.at[pl.ds(off, CH)])
            return carry

        lax.fori_loop(0, n_ch, chunk_body, 0)

    return emb_kernel


def kernel(tokens, table):
    B = tokens.size
    idx = tokens.reshape(B).astype(jnp.int32)
    info = plsc.get_sparse_core_info()
    emb = _emb_call(B, info.num_cores, info.num_subcores)
    out = emb(idx, table)
    return out.reshape(tokens.shape + (EMBED,))


# speedup vs baseline: 1.4842x; 1.0611x over previous
import functools
import math

import jax
import jax.numpy as jnp
from jax import lax
from jax.experimental import pallas as pl
from jax.experimental.pallas import tpu as pltpu
from jax.experimental.pallas import tpu_sc as plsc

EMBED = 32
SCALE = math.sqrt(float(EMBED))
LANES = 16
NB = 8
LA = 4


def _emb_call(S, T, NC, NS):
    NW = NC * NS
    rows_per_w = S // NW
    n_outer = rows_per_w // NB
    assert n_outer * NB == rows_per_w

    mesh = plsc.VectorSubcoreMesh(core_axis_name="c", subcore_axis_name="s")

    @functools.partial(
        pl.kernel,
        mesh=mesh,
        out_type=jax.ShapeDtypeStruct((S, T, EMBED), jnp.float32),
        compiler_params=pltpu.CompilerParams(use_tc_tiling_on_sc=False),
        scratch_types=[
            pltpu.VMEM((rows_per_w, T), jnp.int32),
            pltpu.VMEM((NB, T, EMBED), jnp.float32),
        ]
        + [pltpu.SemaphoreType.DMA] * (2 * NB),
    )
    def emb_kernel(tok_hbm, table_hbm, out_hbm, idx_v, rows_v, *sems):
        g_sems, w_sems = sems[:NB], sems[NB:]
        wid = lax.axis_index("s") * NC + lax.axis_index("c")
        row0 = wid * rows_per_w
        pltpu.sync_copy(tok_hbm.at[pl.ds(row0, rows_per_w)], idx_v)

        def gather(row, b):
            pltpu.async_copy(table_hbm.at[idx_v.at[row]], rows_v.at[b], g_sems[b])

        def wait_gather(b):
            pltpu.make_async_copy(table_hbm.at[idx_v.at[0]], rows_v.at[b], g_sems[b]).wait()

        def writeback(row, b):
            pltpu.async_copy(rows_v.at[b], out_hbm.at[row0 + row], w_sems[b])

        def wait_writeback(b):
            pltpu.make_async_copy(rows_v.at[b], out_hbm.at[row0], w_sems[b]).wait()

        def scale(b):
            def body(i, carry):
                for j in range(EMBED // LANES):
                    sl = pl.ds(j * LANES, LANES)
                    rows_v[b, i, sl] = rows_v[b, i, sl] * SCALE
                return carry

            lax.fori_loop(0, T, body, 0, unroll=4)

        for r in range(LA):
            gather(r, r)

        def outer(o, carry):
            base = o * NB
            for u in range(NB):
                row = base + u
                pb = (u + LA) % NB
                pf_row = row + LA

                @pl.when(pf_row < rows_per_w)
                def _():
                    @pl.when(pf_row >= NB)
                    def _():
                        wait_writeback(pb)

                    gather(pf_row, pb)

                wait_gather(u)
                scale(u)
                writeback(row, u)
            return carry

        lax.fori_loop(0, n_outer, outer, 0)
        for b in range(NB):
            wait_writeback(b)

    return emb_kernel


def kernel(tokens, table):
    S, T = tokens.shape
    if tokens.dtype != jnp.int32:
        tokens = tokens.astype(jnp.int32)
    info = plsc.get_sparse_core_info()
    emb = _emb_call(S, T, info.num_cores, info.num_subcores)
    return emb(tokens, table)
